# baseline (speedup 1.0000x reference)
import functools

import jax
import jax.numpy as jnp
from jax.experimental import pallas as pl
from jax.experimental.pallas import tpu as pltpu

_G = 1024


def _fused_kernel(batch_ref, h_ref, wp_ref, bp_ref, ws_ref, out_ref,
                  acc_ref, den_ref, *, block_rows, win):
    pid = pl.program_id(0)
    nblk = pl.num_programs(0)

    @pl.when(pid == 0)
    def _init():
        acc_ref[:] = jnp.zeros_like(acc_ref)
        den_ref[:] = jnp.zeros_like(den_ref)

    hb = h_ref[:]
    ws = ws_ref[:]
    z = jnp.tanh(
        jax.lax.dot_general(hb, wp_ref[:], (((1,), (1,)), ((), ())),
                            preferred_element_type=jnp.float32)
        + bp_ref[:])
    e_row = jax.lax.dot_general(ws, z, (((1,), (1,)), ((), ())),
                                preferred_element_type=jnp.float32)
    shift = jnp.sum(jnp.abs(ws))
    a_row = jnp.exp(e_row - shift)

    ids_row = batch_ref[0]
    j0 = batch_ref[0, 0, 0] // win
    j1 = batch_ref[0, 0, block_rows - 1] // win
    iota = jax.lax.broadcasted_iota(jnp.int32, (win, block_rows), 0)

    def scatter_window(j):
        base = j * win
        oa = ((ids_row - base) == iota).astype(jnp.float32) * a_row
        part = jax.lax.dot_general(oa, hb, (((1,), (0,)), ((), ())),
                                   preferred_element_type=jnp.float32)
        dpart = jnp.sum(oa, axis=1, keepdims=True)
        acc_ref[pl.ds(base, win), :] += part
        den_ref[pl.ds(base, win), :] += dpart

    scatter_window(j0)

    def body(j, carry):
        scatter_window(j)
        return carry

    jax.lax.fori_loop(j0 + 1, j1 + 1, body, 0)

    @pl.when(pid == nblk - 1)
    def _finish():
        out_ref[:] = acc_ref[:] / jnp.clip(den_ref[:], 1e-12, None)


def kernel(H, batch, Wp, bp, ws):
    V, d = H.shape
    hs = Wp.shape[0]
    block_rows = 16000
    win = 128
    nblk = V // block_rows
    assert nblk * block_rows == V

    batch_i = batch.astype(jnp.int32).reshape(nblk, 1, block_rows)
    bp2 = bp.reshape(1, hs)

    out = pl.pallas_call(
        functools.partial(_fused_kernel, block_rows=block_rows, win=win),
        grid=(nblk,),
        in_specs=[
            pl.BlockSpec((1, 1, block_rows), lambda i: (i, 0, 0)),
            pl.BlockSpec((block_rows, d), lambda i: (i, 0)),
            pl.BlockSpec((hs, d), lambda i: (0, 0)),
            pl.BlockSpec((1, hs), lambda i: (0, 0)),
            pl.BlockSpec((1, hs), lambda i: (0, 0)),
        ],
        out_specs=pl.BlockSpec((_G, d), lambda i: (0, 0)),
        out_shape=jax.ShapeDtypeStruct((_G, d), jnp.float32),
        scratch_shapes=[
            pltpu.VMEM((_G, d), jnp.float32),
            pltpu.VMEM((_G, 1), jnp.float32),
        ],
        compiler_params=pltpu.CompilerParams(
            dimension_semantics=("arbitrary",)),
    )(batch_i, H, Wp, bp2, ws)
    return out


# speedup vs baseline: 1.1191x; 1.1191x over previous
import functools

import jax
import jax.numpy as jnp
from jax.experimental import pallas as pl
from jax.experimental.pallas import tpu as pltpu

_G = 1024


def _fused_kernel(batch_ref, h_ref, wp_ref, bp_ref, ws_ref, out_ref,
                  acc_ref, den_ref, *, block_rows, win):
    pid = pl.program_id(0)
    nblk = pl.num_programs(0)

    @pl.when(pid == 0)
    def _init():
        acc_ref[:] = jnp.zeros_like(acc_ref)
        den_ref[:] = jnp.zeros_like(den_ref)

    hb = h_ref[:]
    hb16 = hb.astype(jnp.bfloat16)
    ws = ws_ref[:]
    z = jnp.tanh(
        jax.lax.dot_general(hb16, wp_ref[:].astype(jnp.bfloat16),
                            (((1,), (1,)), ((), ())),
                            preferred_element_type=jnp.float32)
        + bp_ref[:])
    e_row = jax.lax.dot_general(ws.astype(jnp.bfloat16),
                                z.astype(jnp.bfloat16),
                                (((1,), (1,)), ((), ())),
                                preferred_element_type=jnp.float32)
    shift = jnp.sum(jnp.abs(ws))
    a_row = jnp.exp(e_row - shift)

    ids_row = batch_ref[0]
    j0 = batch_ref[0, 0, 0] // win
    j1 = batch_ref[0, 0, block_rows - 1] // win
    iota = jax.lax.broadcasted_iota(jnp.int32, (win, block_rows), 0)

    def scatter_window(j):
        base = j * win
        oa = ((ids_row - base) == iota).astype(jnp.float32) * a_row
        part = jax.lax.dot_general(oa.astype(jnp.bfloat16), hb16,
                                   (((1,), (0,)), ((), ())),
                                   preferred_element_type=jnp.float32)
        dpart = jnp.sum(oa, axis=1, keepdims=True)
        acc_ref[pl.ds(base, win), :] += part
        den_ref[pl.ds(base, win), :] += dpart

    scatter_window(j0)

    def body(j, carry):
        scatter_window(j)
        return carry

    jax.lax.fori_loop(j0 + 1, j1 + 1, body, 0)

    @pl.when(pid == nblk - 1)
    def _finish():
        out_ref[:] = acc_ref[:] / jnp.clip(den_ref[:], 1e-12, None)


def kernel(H, batch, Wp, bp, ws):
    V, d = H.shape
    hs = Wp.shape[0]
    block_rows = 16000
    win = 64
    nblk = V // block_rows
    assert nblk * block_rows == V

    batch_i = batch.astype(jnp.int32).reshape(nblk, 1, block_rows)
    bp2 = bp.reshape(1, hs)

    out = pl.pallas_call(
        functools.partial(_fused_kernel, block_rows=block_rows, win=win),
        grid=(nblk,),
        in_specs=[
            pl.BlockSpec((1, 1, block_rows), lambda i: (i, 0, 0)),
            pl.BlockSpec((block_rows, d), lambda i: (i, 0)),
            pl.BlockSpec((hs, d), lambda i: (0, 0)),
            pl.BlockSpec((1, hs), lambda i: (0, 0)),
            pl.BlockSpec((1, hs), lambda i: (0, 0)),
        ],
        out_specs=pl.BlockSpec((_G, d), lambda i: (0, 0)),
        out_shape=jax.ShapeDtypeStruct((_G, d), jnp.float32),
        scratch_shapes=[
            pltpu.VMEM((_G, d), jnp.float32),
            pltpu.VMEM((_G, 1), jnp.float32),
        ],
        compiler_params=pltpu.CompilerParams(
            dimension_semantics=("arbitrary",)),
    )(batch_i, H, Wp, bp2, ws)
    return out
